# baseline (speedup 1.0000x reference)
import jax
import jax.numpy as jnp
from jax.experimental import pallas as pl
from jax.experimental.pallas import tpu as pltpu

_N = 10000
_D = 256
_BM = 400
_NJ = _N // _BM
_BE = 400
_NE = _N // _BE
_STEPS = _NE + _NJ
_NBUF = 3


def _layer_norm(h, g, b, eps=1e-5):
    mu = jnp.mean(h, axis=-1, keepdims=True)
    var = jnp.mean((h - mu) ** 2, axis=-1, keepdims=True)
    return (h - mu) / jnp.sqrt(var + eps) * g + b


def _fused_body(x_ref, inc_hbm,
                ew1_ref, eb1_ref, eg_ref, ebe_ref, ew2_ref, eb2_ref, cw_ref,
                dw1_ref, db1_ref, dg_ref, dbe_ref, dw2_ref, db2_ref,
                out_ref, xm_ref, buf_ref, sem_ref):
    i = pl.program_id(0)

    def slab_copy(s):
        return pltpu.make_async_copy(
            inc_hbm.at[pl.ds(s * _BM, _BM), :],
            buf_ref.at[jax.lax.rem(s, _NBUF)],
            sem_ref.at[jax.lax.rem(s, _NBUF)],
        )

    @pl.when(i < _NE)
    def _encode():
        h = jnp.dot(x_ref[...], ew1_ref[...],
                    preferred_element_type=jnp.float32)
        h = jnp.maximum(h + eb1_ref[...], 0.0)
        h = _layer_norm(h, eg_ref[...], ebe_ref[...])
        h = jnp.dot(h, ew2_ref[...], preferred_element_type=jnp.float32)
        h = jnp.maximum(h + eb2_ref[...], 0.0)
        xm_ref[pl.ds(i * _BE, _BE), :] = jnp.dot(
            h, cw_ref[...], preferred_element_type=jnp.float32
        ).astype(jnp.bfloat16)

    @pl.when(i < _NBUF - 1)
    def _prefetch():
        slab_copy(i).start()

    @pl.when(i >= _NE)
    def _conv():
        s = i - _NE

        @pl.when(s + _NBUF - 1 < _NJ)
        def _():
            slab_copy(s + _NBUF - 1).start()

        slab_copy(s).wait()
        blk = buf_ref[jax.lax.rem(s, _NBUF)]
        acc = jnp.dot(blk.astype(jnp.bfloat16), xm_ref[...],
                      preferred_element_type=jnp.float32)
        rs = jnp.sum(blk, axis=1, keepdims=True)
        xt = acc / rs
        h = jnp.dot(xt, dw1_ref[...], preferred_element_type=jnp.float32)
        h = jnp.maximum(h + db1_ref[...], 0.0)
        h = _layer_norm(h, dg_ref[...], dbe_ref[...])
        h = jnp.dot(h, dw2_ref[...], preferred_element_type=jnp.float32)
        out_ref[...] = jnp.maximum(h + db2_ref[...], 0.0)


def kernel(x, incidence, enc_W1, enc_b1, enc_g, enc_be, enc_W2, enc_b2,
           conv_W, dec_W1, dec_b1, dec_g, dec_be, dec_W2, dec_b2):
    f32 = jnp.float32
    ew1, ew2 = enc_W1.T, enc_W2.T
    dw1, dw2 = dec_W1.T, dec_W2.T
    eb1, eb2 = enc_b1.reshape(1, _D), enc_b2.reshape(1, _D)
    db1, db2 = dec_b1.reshape(1, _D), dec_b2.reshape(1, _D)
    eg, ebe = enc_g.reshape(1, _D), enc_be.reshape(1, _D)
    dg, dbe = dec_g.reshape(1, _D), dec_be.reshape(1, _D)

    wspec = pl.BlockSpec((_D, _D), lambda i: (0, 0))
    vspec = pl.BlockSpec((1, _D), lambda i: (0, 0))
    out = pl.pallas_call(
        _fused_body,
        grid=(_STEPS,),
        in_specs=[pl.BlockSpec((_BE, _D), lambda i: (jnp.minimum(i, _NE - 1), 0)),
                  pl.BlockSpec(memory_space=pltpu.MemorySpace.HBM),
                  wspec, vspec, vspec, vspec, wspec, vspec, wspec,
                  wspec, vspec, vspec, vspec, wspec, vspec],
        out_specs=pl.BlockSpec((_BM, _D), lambda i: (jnp.maximum(i - _NE, 0), 0)),
        out_shape=jax.ShapeDtypeStruct((_N, _D), f32),
        scratch_shapes=[pltpu.VMEM((_N, _D), jnp.bfloat16),
                        pltpu.VMEM((_NBUF, _BM, _N), f32),
                        pltpu.SemaphoreType.DMA((_NBUF,))],
    )(x, incidence, ew1, eb1, eg, ebe, ew2, eb2, conv_W,
      dw1, db1, dg, dbe, dw2, db2)
    return out


# speedup vs baseline: 1.0184x; 1.0184x over previous
import jax
import jax.numpy as jnp
from jax.experimental import pallas as pl
from jax.experimental.pallas import tpu as pltpu

_N = 10000
_D = 256
_BM = 200
_NJ = _N // _BM
_BE = 400
_NE = _N // _BE
_STEPS = _NE + _NJ
_NBUF = 6


def _layer_norm(h, g, b, eps=1e-5):
    mu = jnp.mean(h, axis=-1, keepdims=True)
    var = jnp.mean((h - mu) ** 2, axis=-1, keepdims=True)
    return (h - mu) / jnp.sqrt(var + eps) * g + b


def _fused_body(x_ref, inc_hbm,
                ew1_ref, eb1_ref, eg_ref, ebe_ref, ew2_ref, eb2_ref, cw_ref,
                dw1_ref, db1_ref, dg_ref, dbe_ref, dw2_ref, db2_ref,
                out_ref, xm_ref, buf_ref, sem_ref):
    i = pl.program_id(0)

    def slab_copy(s):
        return pltpu.make_async_copy(
            inc_hbm.at[pl.ds(s * _BM, _BM), :],
            buf_ref.at[jax.lax.rem(s, _NBUF)],
            sem_ref.at[jax.lax.rem(s, _NBUF)],
        )

    @pl.when(i < _NE)
    def _encode():
        h = jnp.dot(x_ref[...], ew1_ref[...],
                    preferred_element_type=jnp.float32)
        h = jnp.maximum(h + eb1_ref[...], 0.0)
        h = _layer_norm(h, eg_ref[...], ebe_ref[...])
        h = jnp.dot(h, ew2_ref[...], preferred_element_type=jnp.float32)
        h = jnp.maximum(h + eb2_ref[...], 0.0)
        xm_ref[pl.ds(i * _BE, _BE), :] = jnp.dot(
            h, cw_ref[...], preferred_element_type=jnp.float32
        ).astype(jnp.bfloat16)

    @pl.when(i < _NBUF - 1)
    def _prefetch():
        slab_copy(i).start()

    @pl.when(i >= _NE)
    def _conv():
        s = i - _NE

        @pl.when(s + _NBUF - 1 < _NJ)
        def _():
            slab_copy(s + _NBUF - 1).start()

        slab_copy(s).wait()
        blk = buf_ref[jax.lax.rem(s, _NBUF)]
        acc = jnp.dot(blk.astype(jnp.bfloat16), xm_ref[...],
                      preferred_element_type=jnp.float32)
        rs = jnp.sum(blk, axis=1, keepdims=True)
        xt = acc / rs
        h = jnp.dot(xt, dw1_ref[...], preferred_element_type=jnp.float32)
        h = jnp.maximum(h + db1_ref[...], 0.0)
        h = _layer_norm(h, dg_ref[...], dbe_ref[...])
        h = jnp.dot(h, dw2_ref[...], preferred_element_type=jnp.float32)
        out_ref[...] = jnp.maximum(h + db2_ref[...], 0.0)


def kernel(x, incidence, enc_W1, enc_b1, enc_g, enc_be, enc_W2, enc_b2,
           conv_W, dec_W1, dec_b1, dec_g, dec_be, dec_W2, dec_b2):
    f32 = jnp.float32
    ew1, ew2 = enc_W1.T, enc_W2.T
    dw1, dw2 = dec_W1.T, dec_W2.T
    eb1, eb2 = enc_b1.reshape(1, _D), enc_b2.reshape(1, _D)
    db1, db2 = dec_b1.reshape(1, _D), dec_b2.reshape(1, _D)
    eg, ebe = enc_g.reshape(1, _D), enc_be.reshape(1, _D)
    dg, dbe = dec_g.reshape(1, _D), dec_be.reshape(1, _D)

    wspec = pl.BlockSpec((_D, _D), lambda i: (0, 0))
    vspec = pl.BlockSpec((1, _D), lambda i: (0, 0))
    out = pl.pallas_call(
        _fused_body,
        grid=(_STEPS,),
        in_specs=[pl.BlockSpec((_BE, _D), lambda i: (jnp.minimum(i, _NE - 1), 0)),
                  pl.BlockSpec(memory_space=pltpu.MemorySpace.HBM),
                  wspec, vspec, vspec, vspec, wspec, vspec, wspec,
                  wspec, vspec, vspec, vspec, wspec, vspec],
        out_specs=pl.BlockSpec((_BM, _D), lambda i: (jnp.maximum(i - _NE, 0), 0)),
        out_shape=jax.ShapeDtypeStruct((_N, _D), f32),
        scratch_shapes=[pltpu.VMEM((_N, _D), jnp.bfloat16),
                        pltpu.VMEM((_NBUF, _BM, _N), f32),
                        pltpu.SemaphoreType.DMA((_NBUF,))],
    )(x, incidence, ew1, eb1, eg, ebe, ew2, eb2, conv_W,
      dw1, db1, dg, dbe, dw2, db2)
    return out


# speedup vs baseline: 1.1482x; 1.1275x over previous
import jax
import jax.numpy as jnp
from jax.experimental import pallas as pl
from jax.experimental.pallas import tpu as pltpu

_N = 10000
_D = 256
_BM = 200
_NJ = _N // _BM
_BE = 2000
_NE = _N // _BE
_STEPS = _NE + _NJ
_NBUF = 5


def _layer_norm(h, g, b, eps=1e-5):
    mu = jnp.mean(h, axis=-1, keepdims=True)
    var = jnp.mean((h - mu) ** 2, axis=-1, keepdims=True)
    return (h - mu) / jnp.sqrt(var + eps) * g + b


def _fused_body(x_ref, inc_hbm,
                ew1_ref, eb1_ref, eg_ref, ebe_ref, ew2_ref, eb2_ref, cw_ref,
                dw1_ref, db1_ref, dg_ref, dbe_ref, dw2_ref, db2_ref,
                out_ref, xm_ref, buf_ref, sem_ref):
    i = pl.program_id(0)

    def slab_copy(s):
        return pltpu.make_async_copy(
            inc_hbm.at[pl.ds(s * _BM, _BM), :],
            buf_ref.at[jax.lax.rem(s, _NBUF)],
            sem_ref.at[jax.lax.rem(s, _NBUF)],
        )

    @pl.when(i < _NE)
    def _encode():
        h = jnp.dot(x_ref[...], ew1_ref[...],
                    preferred_element_type=jnp.float32)
        h = jnp.maximum(h + eb1_ref[...], 0.0)
        h = _layer_norm(h, eg_ref[...], ebe_ref[...])
        h = jnp.dot(h, ew2_ref[...], preferred_element_type=jnp.float32)
        h = jnp.maximum(h + eb2_ref[...], 0.0)
        xm_ref[pl.ds(i * _BE, _BE), :] = jnp.dot(
            h, cw_ref[...], preferred_element_type=jnp.float32
        ).astype(jnp.bfloat16)

    @pl.when(i < _NBUF - 1)
    def _prefetch():
        slab_copy(i).start()

    @pl.when(i >= _NE)
    def _conv():
        s = i - _NE

        @pl.when(s + _NBUF - 1 < _NJ)
        def _():
            slab_copy(s + _NBUF - 1).start()

        slab_copy(s).wait()
        blk = buf_ref[jax.lax.rem(s, _NBUF)]
        acc = jnp.dot(blk.astype(jnp.bfloat16), xm_ref[...],
                      preferred_element_type=jnp.float32)
        rs = jnp.sum(blk, axis=1, keepdims=True)
        xt = acc / rs
        h = jnp.dot(xt, dw1_ref[...], preferred_element_type=jnp.float32)
        h = jnp.maximum(h + db1_ref[...], 0.0)
        h = _layer_norm(h, dg_ref[...], dbe_ref[...])
        h = jnp.dot(h, dw2_ref[...], preferred_element_type=jnp.float32)
        out_ref[...] = jnp.maximum(h + db2_ref[...], 0.0)


def kernel(x, incidence, enc_W1, enc_b1, enc_g, enc_be, enc_W2, enc_b2,
           conv_W, dec_W1, dec_b1, dec_g, dec_be, dec_W2, dec_b2):
    f32 = jnp.float32
    ew1, ew2 = enc_W1.T, enc_W2.T
    dw1, dw2 = dec_W1.T, dec_W2.T
    eb1, eb2 = enc_b1.reshape(1, _D), enc_b2.reshape(1, _D)
    db1, db2 = dec_b1.reshape(1, _D), dec_b2.reshape(1, _D)
    eg, ebe = enc_g.reshape(1, _D), enc_be.reshape(1, _D)
    dg, dbe = dec_g.reshape(1, _D), dec_be.reshape(1, _D)

    wspec = pl.BlockSpec((_D, _D), lambda i: (0, 0))
    vspec = pl.BlockSpec((1, _D), lambda i: (0, 0))
    out = pl.pallas_call(
        _fused_body,
        grid=(_STEPS,),
        in_specs=[pl.BlockSpec((_BE, _D), lambda i: (jnp.minimum(i, _NE - 1), 0)),
                  pl.BlockSpec(memory_space=pltpu.MemorySpace.HBM),
                  wspec, vspec, vspec, vspec, wspec, vspec, wspec,
                  wspec, vspec, vspec, vspec, wspec, vspec],
        out_specs=pl.BlockSpec((_BM, _D), lambda i: (jnp.maximum(i - _NE, 0), 0)),
        out_shape=jax.ShapeDtypeStruct((_N, _D), f32),
        scratch_shapes=[pltpu.VMEM((_N, _D), jnp.bfloat16),
                        pltpu.VMEM((_NBUF, _BM, _N), f32),
                        pltpu.SemaphoreType.DMA((_NBUF,))],
    )(x, incidence, ew1, eb1, eg, ebe, ew2, eb2, conv_W,
      dw1, db1, dg, dbe, dw2, db2)
    return out
